# baseline (speedup 1.0000x reference)
import jax
import jax.numpy as jnp
from jax import lax
from jax.experimental import pallas as pl
from jax.experimental.pallas import tpu as pltpu
from jax.experimental.pallas import tpu_sc as plsc

B = 16384
D = 64
NW = 32
BPW = B // NW
PCH = 256
NP = BPW // PCH
L = 16
NC = D // L


def _body(h_hbm, r_hbm, t_hbm, ent_hbm, rel_hbm, out_hbm,
          hi, ti, ri,
          hb0, tb0, rb0, hb1, tb1, rb1, out_v, sem0, sem1):
    cid = lax.axis_index("c")
    sid = lax.axis_index("s")
    wid = sid * 2 + cid
    base = wid * BPW

    pltpu.sync_copy(h_hbm.at[pl.ds(base, BPW)], hi)
    pltpu.sync_copy(t_hbm.at[pl.ds(base, BPW)], ti)
    pltpu.sync_copy(r_hbm.at[pl.ds(base, BPW)], ri)

    hb = (hb0, hb1)
    tb = (tb0, tb1)
    rb = (rb0, rb1)
    sems = (sem0, sem1)

    def fire(p, slot):
        s = pl.ds(p * PCH, PCH)
        return (
            pltpu.async_copy(ent_hbm.at[hi.at[s]], hb[slot], sems[slot]),
            pltpu.async_copy(ent_hbm.at[ti.at[s]], tb[slot], sems[slot]),
            pltpu.async_copy(rel_hbm.at[ri.at[s]], rb[slot], sems[slot]),
        )

    lanes = lax.broadcasted_iota(jnp.int32, (L,), 0)
    last = lanes == (L - 1)
    zeros_i = jnp.zeros((L,), jnp.int32)

    def score(p, slot):
        hs, ts, rs = hb[slot], tb[slot], rb[slot]

        def item(i, _):
            acc = jnp.zeros((L,), jnp.float32)
            for k in range(NC):
                s = pl.ds(k * L, L)
                hv = hs[i, s]
                tv = ts[i, s]
                rv = rs[i, s]
                acc = acc + jnp.abs(hv + rv - tv)
            cs = plsc.cumsum(acc)
            iv = zeros_i + (p * PCH + i)
            plsc.store_scatter(out_v, [iv], -cs, mask=last)
            return 0

        lax.fori_loop(0, PCH, item, 0)

    cps = fire(0, 0)
    for p in range(NP):
        slot = p % 2
        if p + 1 < NP:
            nxt = fire(p + 1, 1 - slot)
        for cp in cps:
            cp.wait()
        score(p, slot)
        if p + 1 < NP:
            cps = nxt

    pltpu.sync_copy(out_v, out_hbm.at[pl.ds(base, BPW)])


@jax.jit
def kernel(h, r, t, entity_table, relation_table):
    k = pl.kernel(
        _body,
        mesh=plsc.VectorSubcoreMesh(core_axis_name="c", subcore_axis_name="s"),
        out_type=jax.ShapeDtypeStruct((B,), jnp.float32),
        compiler_params=pltpu.CompilerParams(
            needs_layout_passes=False,
            use_tc_tiling_on_sc=False,
        ),
        scratch_types=[
            pltpu.VMEM((BPW,), jnp.int32),
            pltpu.VMEM((BPW,), jnp.int32),
            pltpu.VMEM((BPW,), jnp.int32),
            pltpu.VMEM((PCH, D), jnp.float32),
            pltpu.VMEM((PCH, D), jnp.float32),
            pltpu.VMEM((PCH, D), jnp.float32),
            pltpu.VMEM((PCH, D), jnp.float32),
            pltpu.VMEM((PCH, D), jnp.float32),
            pltpu.VMEM((PCH, D), jnp.float32),
            pltpu.VMEM((BPW,), jnp.float32),
            pltpu.SemaphoreType.DMA,
            pltpu.SemaphoreType.DMA,
        ],
    )
    return k(h, r, t, entity_table, relation_table)


# speedup vs baseline: 1.0017x; 1.0017x over previous
import jax
import jax.numpy as jnp
from jax import lax
from jax.experimental import pallas as pl
from jax.experimental.pallas import tpu as pltpu
from jax.experimental.pallas import tpu_sc as plsc

B = 16384
D = 64
NW = 32
BPW = B // NW
PCH = 256
NP = BPW // PCH
L = 16
NC = D // L
NS = 4
CS = PCH // NS


def _body(h_hbm, r_hbm, t_hbm, ent_hbm, rel_hbm, out_hbm,
          hi, ti, ri,
          hb0, tb0, rb0, hb1, tb1, rb1, out_v, sem0, sem1):
    cid = lax.axis_index("c")
    sid = lax.axis_index("s")
    wid = sid * 2 + cid
    base = wid * BPW

    pltpu.sync_copy(h_hbm.at[pl.ds(base, BPW)], hi)
    pltpu.sync_copy(t_hbm.at[pl.ds(base, BPW)], ti)
    pltpu.sync_copy(r_hbm.at[pl.ds(base, BPW)], ri)

    hb = (hb0, hb1)
    tb = (tb0, tb1)
    rb = (rb0, rb1)
    sems = (sem0, sem1)

    def fire(p, slot):
        cps = []
        for j in range(NS):
            s = pl.ds(p * PCH + j * CS, CS)
            d = pl.ds(j * CS, CS)
            cps.append(pltpu.async_copy(
                ent_hbm.at[hi.at[s]], hb[slot].at[d], sems[slot]))
            cps.append(pltpu.async_copy(
                ent_hbm.at[ti.at[s]], tb[slot].at[d], sems[slot]))
            cps.append(pltpu.async_copy(
                rel_hbm.at[ri.at[s]], rb[slot].at[d], sems[slot]))
        return cps

    lanes = lax.broadcasted_iota(jnp.int32, (L,), 0)
    last = lanes == (L - 1)
    zeros_i = jnp.zeros((L,), jnp.int32)

    def score(p, slot):
        hs, ts, rs = hb[slot], tb[slot], rb[slot]

        def item(i, _):
            acc = jnp.zeros((L,), jnp.float32)
            for k in range(NC):
                s = pl.ds(k * L, L)
                hv = hs[i, s]
                tv = ts[i, s]
                rv = rs[i, s]
                acc = acc + jnp.abs(hv + rv - tv)
            cs = plsc.cumsum(acc)
            iv = zeros_i + (p * PCH + i)
            plsc.store_scatter(out_v, [iv], -cs, mask=last)
            return 0

        lax.fori_loop(0, PCH, item, 0)

    cps = fire(0, 0)
    for p in range(NP):
        slot = p % 2
        if p + 1 < NP:
            nxt = fire(p + 1, 1 - slot)
        for cp in cps:
            cp.wait()
        score(p, slot)
        if p + 1 < NP:
            cps = nxt

    pltpu.sync_copy(out_v, out_hbm.at[pl.ds(base, BPW)])


@jax.jit
def kernel(h, r, t, entity_table, relation_table):
    k = pl.kernel(
        _body,
        mesh=plsc.VectorSubcoreMesh(core_axis_name="c", subcore_axis_name="s"),
        out_type=jax.ShapeDtypeStruct((B,), jnp.float32),
        compiler_params=pltpu.CompilerParams(
            needs_layout_passes=False,
            use_tc_tiling_on_sc=False,
        ),
        scratch_types=[
            pltpu.VMEM((BPW,), jnp.int32),
            pltpu.VMEM((BPW,), jnp.int32),
            pltpu.VMEM((BPW,), jnp.int32),
            pltpu.VMEM((PCH, D), jnp.float32),
            pltpu.VMEM((PCH, D), jnp.float32),
            pltpu.VMEM((PCH, D), jnp.float32),
            pltpu.VMEM((PCH, D), jnp.float32),
            pltpu.VMEM((PCH, D), jnp.float32),
            pltpu.VMEM((PCH, D), jnp.float32),
            pltpu.VMEM((BPW,), jnp.float32),
            pltpu.SemaphoreType.DMA,
            pltpu.SemaphoreType.DMA,
        ],
    )
    return k(h, r, t, entity_table, relation_table)
